# baseline (speedup 1.0000x reference)
import jax
import jax.numpy as jnp
from jax import lax
from jax.experimental import pallas as pl
from jax.experimental.pallas import tpu as pltpu
from jax.experimental.pallas import tpu_sc as plsc

B = 16384
V = 1000
D = 1000
DP = 1024
NC = 2
NS = 16
NW = NC * NS
BPW = B // NW
CHUNK = 32
NCH = BPW // CHUNK
RB = 512


def _sigmoid_pad_body(x_ref, o_ref):
    o_ref[:, :D] = jax.nn.sigmoid(x_ref[...])
    o_ref[:, D:] = jnp.zeros((V, DP - D), jnp.float32)


def _sigmoid_table(logits):
    return pl.pallas_call(
        _sigmoid_pad_body,
        out_shape=jax.ShapeDtypeStruct((V, DP), jnp.float32),
    )(logits)


NBUF = 3


def _gather_body(s_hbm, idx_hbm, out_hbm, idx_v, *rest):
    bufs = rest[:NBUF]
    sin = rest[NBUF:2 * NBUF]
    sout = rest[2 * NBUF:3 * NBUF]
    wid = lax.axis_index("s") * NC + lax.axis_index("c")
    base = wid * BPW
    pltpu.sync_copy(idx_hbm.at[pl.ds(base, BPW)], idx_v)

    def start_in(j, b):
        return pltpu.async_copy(
            s_hbm.at[idx_v.at[pl.ds(j * CHUNK, CHUNK)]], bufs[b], sin[b])

    def start_out(j, b):
        return pltpu.async_copy(
            bufs[b], out_hbm.at[pl.ds(base + j * CHUNK, CHUNK)], sout[b])

    h_in = [None] * NBUF
    h_out = [None] * NBUF
    h_in[0] = start_in(0, 0)
    for j in range(NCH):
        b = j % NBUF
        h_in[b].wait()
        h_out[b] = start_out(j, b)
        if j + 1 < NCH:
            bn = (j + 1) % NBUF
            if h_out[bn] is not None:
                h_out[bn].wait()
                h_out[bn] = None
            h_in[bn] = start_in(j + 1, bn)
    for h in h_out:
        if h is not None:
            h.wait()


_gather = pl.kernel(
    _gather_body,
    out_type=jax.ShapeDtypeStruct((B, DP), jnp.float32),
    mesh=plsc.VectorSubcoreMesh(core_axis_name="c", subcore_axis_name="s"),
    scratch_types=(
        [pltpu.VMEM((BPW,), jnp.int32)]
        + [pltpu.VMEM((CHUNK, DP), jnp.float32)] * NBUF
        + [pltpu.SemaphoreType.DMA] * (2 * NBUF)
    ),
)


@jax.jit
def kernel(indices, logits):
    s = _sigmoid_table(logits)
    return _gather(s, indices)[:, :D]


# speedup vs baseline: 1.0257x; 1.0257x over previous
import jax
import jax.numpy as jnp
from jax import lax
from jax.experimental import pallas as pl
from jax.experimental.pallas import tpu as pltpu
from jax.experimental.pallas import tpu_sc as plsc

B = 16384
V = 1000
D = 1000
DP = 1024
NC = 2
NS = 16
NW = NC * NS
BPW = B // NW
CHUNK = 16
NCH = BPW // CHUNK
LOOKAHEAD = 2


def _sigmoid_pad_body(x_ref, o_ref):
    o_ref[:, :D] = jax.nn.sigmoid(x_ref[...])
    o_ref[:, D:] = jnp.zeros((V, DP - D), jnp.float32)


def _sigmoid_table(logits):
    return pl.pallas_call(
        _sigmoid_pad_body,
        out_shape=jax.ShapeDtypeStruct((V, DP), jnp.float32),
    )(logits)


NBUF = 4


def _gather_body(s_hbm, idx_hbm, out_hbm, idx_v, *rest):
    bufs = rest[:NBUF]
    sin = rest[NBUF:2 * NBUF]
    sout = rest[2 * NBUF:3 * NBUF]
    wid = lax.axis_index("s") * NC + lax.axis_index("c")
    base = wid * BPW
    pltpu.sync_copy(idx_hbm.at[pl.ds(base, BPW)], idx_v)

    def start_in(j, b):
        return pltpu.async_copy(
            s_hbm.at[idx_v.at[pl.ds(j * CHUNK, CHUNK)]], bufs[b], sin[b])

    def start_out(j, b):
        return pltpu.async_copy(
            bufs[b], out_hbm.at[pl.ds(base + j * CHUNK, CHUNK)], sout[b])

    h_in = [None] * NBUF
    h_out = [None] * NBUF
    for j in range(min(LOOKAHEAD, NCH)):
        h_in[j % NBUF] = start_in(j, j % NBUF)
    for j in range(NCH):
        b = j % NBUF
        h_in[b].wait()
        h_out[b] = start_out(j, b)
        nxt = j + LOOKAHEAD
        if nxt < NCH:
            bn = nxt % NBUF
            if h_out[bn] is not None:
                h_out[bn].wait()
                h_out[bn] = None
            h_in[bn] = start_in(nxt, bn)
    for h in h_out:
        if h is not None:
            h.wait()


_gather = pl.kernel(
    _gather_body,
    out_type=jax.ShapeDtypeStruct((B, DP), jnp.float32),
    mesh=plsc.VectorSubcoreMesh(core_axis_name="c", subcore_axis_name="s"),
    scratch_types=(
        [pltpu.VMEM((BPW,), jnp.int32)]
        + [pltpu.VMEM((CHUNK, DP), jnp.float32)] * NBUF
        + [pltpu.SemaphoreType.DMA] * (2 * NBUF)
    ),
)


@jax.jit
def kernel(indices, logits):
    s = _sigmoid_table(logits)
    return _gather(s, indices)[:, :D]
